# baseline (speedup 1.0000x reference)
import dataclasses
import functools

import jax
import jax.numpy as jnp
from jax import lax
from jax.experimental import pallas as pl
from jax.experimental.pallas import tpu as pltpu
from jax.experimental.pallas import tpu_sc as plsc

N = 10000
E = 320000
D = 128
NP = 10240
NC = 2
NS = 16
NW = NC * NS
CK = 128
CJ = 80
EW = CJ * CK
EP = NW * EW
TRB = 10016
SCJ = 16
NSC = CJ // SCJ
G16 = CK // 16
SLC = NP // NS
BT = 1024

_mesh = plsc.VectorSubcoreMesh(core_axis_name="c", subcore_axis_name="s",
                               num_cores=NC, num_subcores=NS)
_sc_params = pltpu.CompilerParams()
if "needs_layout_passes" in pltpu.CompilerParams.__dataclass_fields__:
    _sc_params = dataclasses.replace(_sc_params, needs_layout_passes=False)
_f32 = jnp.float32
_i32 = jnp.int32


def _ids():
    cid = lax.axis_index("c")
    sid = lax.axis_index("s")
    return cid, sid, cid * NS + sid


def _zero_1d(ref):
    z = jnp.zeros((16,), _f32)

    @pl.loop(0, ref.shape[0], step=16)
    def _(i):
        ref[pl.ds(i, 16)] = z


def _reduce_tiles(shared, k_off, stride, red, stg, rsem, out_h, out_off,
                  sid):
    base = sid * SLC
    for s in range(NS):
        pltpu.async_copy(shared.at[pl.ds(s * stride + k_off + base, SLC)],
                         stg.at[s], rsem)
    for s in range(NS):
        pltpu.make_async_copy(shared.at[pl.ds(k_off + base, SLC)],
                              stg.at[s], rsem).wait()

    @pl.loop(0, SLC, step=16)
    def _(i):
        sl = pl.ds(i, 16)
        acc16 = stg[0, sl]
        for s in range(1, NS):
            acc16 = acc16 + stg[s, sl]
        red[sl] = acc16

    pltpu.sync_copy(red, out_h.at[pl.ds(out_off + base, SLC)])


@functools.partial(
    pl.kernel,
    out_type=jax.ShapeDtypeStruct((NC * 3 * NP,), _f32),
    mesh=_mesh,
    compiler_params=_sc_params,
    scratch_types=[
        pltpu.VMEM((CJ, CK), _i32),
        pltpu.VMEM((CJ, CK), _i32),
        pltpu.VMEM((NP,), _f32),
        pltpu.VMEM((NP,), _f32),
        pltpu.VMEM((NP,), _f32),
        pltpu.VMEM((NP,), _f32),
        pltpu.VMEM_SHARED((NS * 3 * NP,), _f32),
        pltpu.VMEM((NS, SLC), _f32),
        pltpu.VMEM((SLC,), _f32),
        pltpu.SemaphoreType.DMA,
    ],
)
def _sc_stats(row_h, col_h, px_h, out_h, row_v, col_v, px_v,
              acc_d, acc_h, acc_n, shared, stg, red, rsem):
    cid, sid, wid = _ids()
    pltpu.sync_copy(row_h.at[wid], row_v)
    pltpu.sync_copy(col_h.at[wid], col_v)
    pltpu.sync_copy(px_h, px_v)
    _zero_1d(acc_d)
    _zero_1d(acc_h)
    _zero_1d(acc_n)
    ones = jnp.ones((16,), _f32)

    @pl.loop(0, CJ)
    def _(j):
        for g in range(G16):
            sl = pl.ds(g * 16, 16)
            r = row_v[j, sl]
            c = col_v[j, sl]
            p = plsc.load_gather(px_v, [r])
            plsc.addupdate_scatter(acc_h, [c], p)
            plsc.addupdate_scatter(acc_d, [c], ones)
            plsc.addupdate_scatter(acc_n, [r], ones)

    pltpu.sync_copy(acc_d, shared.at[pl.ds((sid * 3 + 0) * NP, NP)])
    pltpu.sync_copy(acc_h, shared.at[pl.ds((sid * 3 + 1) * NP, NP)])
    pltpu.sync_copy(acc_n, shared.at[pl.ds((sid * 3 + 2) * NP, NP)])
    plsc.subcore_barrier()
    for k in range(3):
        _reduce_tiles(shared, k * NP, 3 * NP, red, stg, rsem,
                      out_h, (cid * 3 + k) * NP, sid)


@functools.partial(
    pl.kernel,
    out_type=(jax.ShapeDtypeStruct((NW, CJ, CK), _f32),
              jax.ShapeDtypeStruct((NC * NP,), _f32)),
    mesh=_mesh,
    compiler_params=_sc_params,
    scratch_types=[
        pltpu.VMEM((CJ, CK), _i32),
        pltpu.VMEM((CJ, CK), _i32),
        pltpu.VMEM((CJ, CK), _f32),
        pltpu.VMEM((NP,), _f32),
        pltpu.VMEM((NP,), _f32),
        pltpu.VMEM((NP,), _f32),
        pltpu.VMEM((NP,), _f32),
        pltpu.VMEM_SHARED((NS * NP,), _f32),
        pltpu.VMEM((NS, SLC), _f32),
        pltpu.VMEM((SLC,), _f32),
        pltpu.SemaphoreType.DMA,
    ],
)
def _sc_soft(row_h, col_h, ax_h, parta_h, ea_h, ssum_h, row_v, col_v, ea_v,
             ax_v, ae_v, s1, acc, shared, stg, red, rsem):
    cid, sid, wid = _ids()
    pltpu.sync_copy(row_h.at[wid], row_v)
    pltpu.sync_copy(col_h.at[wid], col_v)
    pltpu.sync_copy(ax_h, ax_v)
    pltpu.sync_copy(parta_h.at[pl.ds(1 * NP, NP)], ae_v)
    pltpu.sync_copy(parta_h.at[pl.ds(4 * NP, NP)], s1)

    @pl.loop(0, NP, step=16)
    def _(i):
        sl = pl.ds(i, 16)
        ae_v[sl] = ae_v[sl] + s1[sl]

    pltpu.sync_copy(parta_h.at[pl.ds(0 * NP, NP)], s1)
    pltpu.sync_copy(parta_h.at[pl.ds(3 * NP, NP)], acc)

    @pl.loop(0, NP, step=16)
    def _(i):
        sl = pl.ds(i, 16)
        d = s1[sl] + acc[sl]
        ae_v[sl] = ae_v[sl] / jnp.maximum(d, 1.0)

    _zero_1d(acc)

    @pl.loop(0, CJ)
    def _(j):
        for g in range(G16):
            sl = pl.ds(g * 16, 16)
            r = row_v[j, sl]
            c = col_v[j, sl]
            a0 = plsc.load_gather(ax_v, [r]) + plsc.load_gather(ae_v, [c])
            a = jnp.where(a0 > 0.0, a0, 0.2 * a0)
            e = jnp.exp(a)
            ea_v[j, sl] = e
            plsc.addupdate_scatter(acc, [c], e)

    pltpu.sync_copy(ea_v, ea_h.at[wid])
    pltpu.sync_copy(acc, shared.at[pl.ds(sid * NP, NP)])
    plsc.subcore_barrier()
    _reduce_tiles(shared, 0, NP, red, stg, rsem, ssum_h, cid * NP, sid)


@functools.partial(
    pl.kernel,
    out_type=(jax.ShapeDtypeStruct((NW, CJ, CK), _f32),
              jax.ShapeDtypeStruct((NW, CJ, CK), _f32)),
    mesh=_mesh,
    compiler_params=_sc_params,
    scratch_types=[
        pltpu.VMEM((CJ, CK), _i32),
        pltpu.VMEM((CJ, CK), _i32),
        pltpu.VMEM((CJ, CK), _f32),
        pltpu.VMEM((CJ, CK), _f32),
        pltpu.VMEM((CJ, CK), _f32),
        pltpu.VMEM((NP,), _f32),
        pltpu.VMEM((NP,), _f32),
        pltpu.VMEM((NP,), _f32),
    ],
)
def _sc_weights(row_h, col_h, ea_h, sb_h, qs_h, dn_h, w1_h, w2_h,
                row_v, col_v, ea_v, w1_v, w2_v, sb_v, qs_v, dn_v):
    cid, sid, wid = _ids()
    pltpu.sync_copy(row_h.at[wid], row_v)
    pltpu.sync_copy(col_h.at[wid], col_v)
    pltpu.sync_copy(ea_h.at[wid], ea_v)
    pltpu.sync_copy(sb_h, sb_v)
    pltpu.sync_copy(qs_h, qs_v)
    pltpu.sync_copy(dn_h, dn_v)

    @pl.loop(0, CJ)
    def _(j):
        for g in range(G16):
            sl = pl.ds(g * 16, 16)
            r = row_v[j, sl]
            c = col_v[j, sl]
            e = ea_v[j, sl]
            w1_v[j, sl] = e * plsc.load_gather(sb_v, [c])
            w2_v[j, sl] = (e * plsc.load_gather(dn_v, [r])
                           * plsc.load_gather(qs_v, [c]))

    pltpu.sync_copy(w1_v, w1_h.at[wid])
    pltpu.sync_copy(w2_v, w2_h.at[wid])


@functools.partial(
    pl.kernel,
    out_type=jax.ShapeDtypeStruct((NC, NP, D), _f32),
    mesh=_mesh,
    compiler_params=_sc_params,
    scratch_types=[
        pltpu.VMEM((SCJ, CK), _i32),
        pltpu.VMEM((SCJ, CK), _i32),
        pltpu.VMEM((SCJ, CK), _f32),
        pltpu.VMEM((CK, D), _f32),
        pltpu.VMEM((CK, D), _f32),
        pltpu.VMEM_SHARED((NP, D), _f32),
        pltpu.SemaphoreType.DMA,
        pltpu.SemaphoreType.DMA,
        pltpu.SemaphoreType.DMA,
        pltpu.SemaphoreType.DMA,
    ],
)
def _sc_prop(gi_h, si_h, w_h, tab_h, out_h,
             gi_v, si_v, w_v, bufa, bufb, acc_sh, gsa, gsb, ssa, ssb):
    cid, sid, wid = _ids()

    z = jnp.zeros((16,), _f32)

    @pl.loop(0, CK)
    def _(rr):
        for k in range(D // 16):
            bufa[rr, pl.ds(k * 16, 16)] = z

    @pl.loop(0, SLC, step=CK)
    def _(k):
        pltpu.sync_copy(bufa, acc_sh.at[pl.ds(sid * SLC + k, CK)])

    plsc.subcore_barrier()

    def scale(buf, jj):
        jsplat = jnp.full((16,), jj, _i32)

        @pl.loop(0, CK)
        def _(rr):
            wv = plsc.load_gather(w_v, [jsplat, jnp.full((16,), rr, _i32)])
            for k in range(D // 16):
                slk = pl.ds(k * 16, 16)
                buf[rr, slk] = buf[rr, slk] * wv

    @pl.loop(0, NSC)
    def _(sc):
        scs = pl.ds(sc * SCJ, SCJ)
        pltpu.sync_copy(gi_h.at[wid, scs], gi_v)
        pltpu.sync_copy(si_h.at[wid, scs], si_v)
        pltpu.sync_copy(w_h.at[wid, scs], w_v)

        pltpu.async_copy(tab_h.at[gi_v.at[0]], bufa, gsa)
        pltpu.async_copy(tab_h.at[gi_v.at[1]], bufb, gsb)

        @pl.loop(0, SCJ, step=2)
        def _(jj):
            pltpu.make_async_copy(tab_h.at[gi_v.at[jj]], bufa, gsa).wait()
            scale(bufa, jj)
            pltpu.async_copy(bufa, acc_sh.at[si_v.at[jj]], ssa, add=True)
            pltpu.make_async_copy(tab_h.at[gi_v.at[jj + 1]], bufb, gsb).wait()
            scale(bufb, jj + 1)
            pltpu.async_copy(bufb, acc_sh.at[si_v.at[jj + 1]], ssb, add=True)

            @pl.when(jj + 2 < SCJ)
            def _():
                pltpu.make_async_copy(bufa, acc_sh.at[si_v.at[jj]], ssa).wait()
                pltpu.async_copy(tab_h.at[gi_v.at[jj + 2]], bufa, gsa)
                pltpu.make_async_copy(bufb, acc_sh.at[si_v.at[jj + 1]],
                                      ssb).wait()
                pltpu.async_copy(tab_h.at[gi_v.at[jj + 3]], bufb, gsb)

        pltpu.make_async_copy(bufa, acc_sh.at[si_v.at[0]], ssa).wait()
        pltpu.make_async_copy(bufb, acc_sh.at[si_v.at[0]], ssb).wait()

    plsc.subcore_barrier()
    pltpu.sync_copy(acc_sh.at[pl.ds(sid * SLC, SLC)],
                    out_h.at[cid, pl.ds(sid * SLC, SLC)])


def _lin_common(xb, w_ref, att_ref, xlin_ref, ax_ref, px_ref):
    W = w_ref[...]
    aF = att_ref[0, :]
    aH = att_ref[1, :]
    xl = lax.dot_general(xb, W, (((1,), (1,)), ((), ())),
                         preferred_element_type=_f32)
    u = jnp.dot(aH, W)
    xlin_ref[...] = xl
    ax_ref[...] = jnp.sum(xl * aF[None, :], axis=1)
    px_ref[...] = jnp.sum(xb * u[None, :], axis=1)


_LIN_OUT_SPECS = [
    pl.BlockSpec((BT, D), lambda i: (i, 0)),
    pl.BlockSpec((BT,), lambda i: (i,)),
    pl.BlockSpec((BT,), lambda i: (i,)),
]
_LIN_OUT_SHAPE = [
    jax.ShapeDtypeStruct((NP, D), _f32),
    jax.ShapeDtypeStruct((NP,), _f32),
    jax.ShapeDtypeStruct((NP,), _f32),
]


def _tc_lin_body(x_ref, w_ref, att_ref, xlin_ref, ax_ref, px_ref):
    _lin_common(x_ref[...], w_ref, att_ref, xlin_ref, ax_ref, px_ref)


def _tc_lin(x_pad, W, att2):
    return pl.pallas_call(
        _tc_lin_body,
        grid=(NP // BT,),
        in_specs=[
            pl.BlockSpec((BT, D), lambda i: (i, 0)),
            pl.BlockSpec((D, D), lambda i: (0, 0)),
            pl.BlockSpec((2, D), lambda i: (0, 0)),
        ],
        out_specs=_LIN_OUT_SPECS,
        out_shape=_LIN_OUT_SHAPE,
    )(x_pad, W, att2)


def _tc_tables_body(pa_ref, ss_ref, sb_ref, qs_ref, dn_ref):
    deg = pa_ref[0, 0] + pa_ref[1, 0]
    dn = pa_ref[0, 2] + pa_ref[1, 2]
    ss = ss_ref[0] + ss_ref[1]
    qs = 1.0 / (ss + 1e-16)
    qs_ref[...] = qs
    sb_ref[...] = jnp.where(deg > 0.0, qs / deg, 0.0)
    dn_ref[...] = jnp.where(dn > 0.0, 1.0 / dn, 0.0)


def _tc_tables(parta, ssum):
    pa = parta.reshape(NC, 3, NP)
    ss = ssum.reshape(NC, NP)
    return pl.pallas_call(
        _tc_tables_body,
        grid=(NP // BT,),
        in_specs=[
            pl.BlockSpec((NC, 3, BT), lambda i: (0, 0, i)),
            pl.BlockSpec((NC, BT), lambda i: (0, i)),
        ],
        out_specs=[pl.BlockSpec((BT,), lambda i: (i,))] * 3,
        out_shape=[jax.ShapeDtypeStruct((NP,), _f32)] * 3,
    )(pa, ss)


def _tc_sum_body(p_ref, o_ref):
    o_ref[...] = p_ref[0] + p_ref[1]


def _tc_sum(part):
    return pl.pallas_call(
        _tc_sum_body,
        grid=(NP // BT,),
        in_specs=[pl.BlockSpec((NC, BT, D), lambda i: (0, i, 0))],
        out_specs=pl.BlockSpec((BT, D), lambda i: (i, 0)),
        out_shape=jax.ShapeDtypeStruct((NP, D), _f32),
    )(part)


def _tc_mid_body(p_ref, b_ref, ap_ref, w_ref, att_ref,
                 xlin_ref, ax_ref, px_ref):
    ap = ap_ref[0, 0]
    t = p_ref[0] + p_ref[1] + b_ref[0, :][None, :]
    xb = jnp.where(t >= 0.0, t, ap * t)
    _lin_common(xb, w_ref, att_ref, xlin_ref, ax_ref, px_ref)


def _tc_mid(part, b, ap, W, att2):
    return pl.pallas_call(
        _tc_mid_body,
        grid=(NP // BT,),
        in_specs=[
            pl.BlockSpec((NC, BT, D), lambda i: (0, i, 0)),
            pl.BlockSpec((1, D), lambda i: (0, 0)),
            pl.BlockSpec((1, 1), lambda i: (0, 0)),
            pl.BlockSpec((D, D), lambda i: (0, 0)),
            pl.BlockSpec((2, D), lambda i: (0, 0)),
        ],
        out_specs=_LIN_OUT_SPECS,
        out_shape=_LIN_OUT_SHAPE,
    )(part, b, ap, W, att2)


def _tc_final_body(p_ref, b_ref, x_ref, ap_ref, o_ref):
    ap = ap_ref[0, 0]
    t = p_ref[0] + p_ref[1] + b_ref[0, :][None, :] + x_ref[...]
    o_ref[...] = jnp.where(t >= 0.0, t, ap * t)


def _tc_final(part, b, x_pad, ap):
    return pl.pallas_call(
        _tc_final_body,
        grid=(NP // BT,),
        in_specs=[
            pl.BlockSpec((NC, BT, D), lambda i: (0, i, 0)),
            pl.BlockSpec((1, D), lambda i: (0, 0)),
            pl.BlockSpec((BT, D), lambda i: (i, 0)),
            pl.BlockSpec((1, 1), lambda i: (0, 0)),
        ],
        out_specs=pl.BlockSpec((BT, D), lambda i: (i, 0)),
        out_shape=jax.ShapeDtypeStruct((NP, D), _f32),
    )(part, b, x_pad, ap)


def kernel(x, edge_index, W1, att1, b1, W2, att2, b2, a_prelu):
    epw = E // NW
    padw = EW - epw
    padi = jnp.broadcast_to(
        (TRB + jnp.arange(NW, dtype=_i32))[:, None], (NW, padw))
    row3 = jnp.concatenate(
        [edge_index[0].reshape(NW, epw), padi], axis=1).reshape(NW, CJ, CK)
    col3 = jnp.concatenate(
        [edge_index[1].reshape(NW, epw), padi], axis=1).reshape(NW, CJ, CK)
    x_pad = jnp.pad(x, ((0, NP - N), (0, 0)))
    att1_2 = att1.reshape(2, D)
    att2_2 = att2.reshape(2, D)
    b1_2 = b1.reshape(1, D)
    b2_2 = b2.reshape(1, D)
    ap = a_prelu.reshape(1, 1)

    def layer(xlin, ax, px):
        parta = _sc_stats(row3, col3, px)
        ea, ssum = _sc_soft(row3, col3, ax, parta)
        sb, qs, dninv = _tc_tables(parta, ssum)
        w1, w2 = _sc_weights(row3, col3, ea, sb, qs, dninv)
        parte = _sc_prop(row3, col3, w1, xlin)
        oute = _tc_sum(parte)
        return _sc_prop(col3, row3, w2, oute)

    xlin1, ax1, px1 = _tc_lin(x_pad, W1, att1_2)
    partn1 = layer(xlin1, ax1, px1)
    xlin2, ax2, px2 = _tc_mid(partn1, b1_2, ap, W2, att2_2)
    partn2 = layer(xlin2, ax2, px2)
    out = _tc_final(partn2, b2_2, x_pad, ap)
    return out[:N]


# speedup vs baseline: 1.0751x; 1.0751x over previous
import dataclasses
import functools

import jax
import jax.numpy as jnp
from jax import lax
from jax.experimental import pallas as pl
from jax.experimental.pallas import tpu as pltpu
from jax.experimental.pallas import tpu_sc as plsc

N = 10000
E = 320000
D = 128
NP = 10240
NC = 2
NS = 16
NW = NC * NS
CK = 128
CJ = 80
EW = CJ * CK
EP = NW * EW
TRB = 10016
SCJ = 16
NSC = CJ // SCJ
CK2 = 64
SCJ2 = 32
NSC2 = EW // (SCJ2 * CK2)
NBUF = 4
G16 = CK // 16
SLC = NP // NS
BT = 1024

_mesh = plsc.VectorSubcoreMesh(core_axis_name="c", subcore_axis_name="s",
                               num_cores=NC, num_subcores=NS)
_sc_params = pltpu.CompilerParams()
if "needs_layout_passes" in pltpu.CompilerParams.__dataclass_fields__:
    _sc_params = dataclasses.replace(_sc_params, needs_layout_passes=False)
_f32 = jnp.float32
_i32 = jnp.int32


def _ids():
    cid = lax.axis_index("c")
    sid = lax.axis_index("s")
    return cid, sid, cid * NS + sid


def _zero_1d(ref):
    z = jnp.zeros((16,), _f32)

    @pl.loop(0, ref.shape[0], step=16)
    def _(i):
        ref[pl.ds(i, 16)] = z


def _reduce_tiles(shared, k_off, stride, red, stg, rsem, out_h, out_off,
                  sid):
    base = sid * SLC
    for s in range(NS):
        pltpu.async_copy(shared.at[pl.ds(s * stride + k_off + base, SLC)],
                         stg.at[s], rsem)
    for s in range(NS):
        pltpu.make_async_copy(shared.at[pl.ds(k_off + base, SLC)],
                              stg.at[s], rsem).wait()

    @pl.loop(0, SLC, step=16)
    def _(i):
        sl = pl.ds(i, 16)
        acc16 = stg[0, sl]
        for s in range(1, NS):
            acc16 = acc16 + stg[s, sl]
        red[sl] = acc16

    pltpu.sync_copy(red, out_h.at[pl.ds(out_off + base, SLC)])


@functools.partial(
    pl.kernel,
    out_type=jax.ShapeDtypeStruct((NC * 3 * NP,), _f32),
    mesh=_mesh,
    compiler_params=_sc_params,
    scratch_types=[
        pltpu.VMEM((CJ, CK), _i32),
        pltpu.VMEM((CJ, CK), _i32),
        pltpu.VMEM((NP,), _f32),
        pltpu.VMEM((NP,), _f32),
        pltpu.VMEM((NP,), _f32),
        pltpu.VMEM((NP,), _f32),
        pltpu.VMEM_SHARED((NS * 3 * NP,), _f32),
        pltpu.VMEM((NS, SLC), _f32),
        pltpu.VMEM((SLC,), _f32),
        pltpu.SemaphoreType.DMA,
    ],
)
def _sc_stats(row_h, col_h, px_h, out_h, row_v, col_v, px_v,
              acc_d, acc_h, acc_n, shared, stg, red, rsem):
    cid, sid, wid = _ids()
    pltpu.sync_copy(row_h.at[wid], row_v)
    pltpu.sync_copy(col_h.at[wid], col_v)
    pltpu.sync_copy(px_h, px_v)
    _zero_1d(acc_d)
    _zero_1d(acc_h)
    _zero_1d(acc_n)
    ones = jnp.ones((16,), _f32)

    @pl.loop(0, CJ)
    def _(j):
        for g in range(G16):
            sl = pl.ds(g * 16, 16)
            r = row_v[j, sl]
            c = col_v[j, sl]
            p = plsc.load_gather(px_v, [r])
            plsc.addupdate_scatter(acc_h, [c], p)
            plsc.addupdate_scatter(acc_d, [c], ones)
            plsc.addupdate_scatter(acc_n, [r], ones)

    pltpu.sync_copy(acc_d, shared.at[pl.ds((sid * 3 + 0) * NP, NP)])
    pltpu.sync_copy(acc_h, shared.at[pl.ds((sid * 3 + 1) * NP, NP)])
    pltpu.sync_copy(acc_n, shared.at[pl.ds((sid * 3 + 2) * NP, NP)])
    plsc.subcore_barrier()
    for k in range(3):
        _reduce_tiles(shared, k * NP, 3 * NP, red, stg, rsem,
                      out_h, (cid * 3 + k) * NP, sid)


@functools.partial(
    pl.kernel,
    out_type=(jax.ShapeDtypeStruct((NW, CJ, CK), _f32),
              jax.ShapeDtypeStruct((NC * NP,), _f32)),
    mesh=_mesh,
    compiler_params=_sc_params,
    scratch_types=[
        pltpu.VMEM((CJ, CK), _i32),
        pltpu.VMEM((CJ, CK), _i32),
        pltpu.VMEM((CJ, CK), _f32),
        pltpu.VMEM((NP,), _f32),
        pltpu.VMEM((NP,), _f32),
        pltpu.VMEM((NP,), _f32),
        pltpu.VMEM((NP,), _f32),
        pltpu.VMEM_SHARED((NS * NP,), _f32),
        pltpu.VMEM((NS, SLC), _f32),
        pltpu.VMEM((SLC,), _f32),
        pltpu.SemaphoreType.DMA,
    ],
)
def _sc_soft(row_h, col_h, ax_h, parta_h, ea_h, ssum_h, row_v, col_v, ea_v,
             ax_v, ae_v, s1, acc, shared, stg, red, rsem):
    cid, sid, wid = _ids()
    pltpu.sync_copy(row_h.at[wid], row_v)
    pltpu.sync_copy(col_h.at[wid], col_v)
    pltpu.sync_copy(ax_h, ax_v)
    pltpu.sync_copy(parta_h.at[pl.ds(1 * NP, NP)], ae_v)
    pltpu.sync_copy(parta_h.at[pl.ds(4 * NP, NP)], s1)

    @pl.loop(0, NP, step=16)
    def _(i):
        sl = pl.ds(i, 16)
        ae_v[sl] = ae_v[sl] + s1[sl]

    pltpu.sync_copy(parta_h.at[pl.ds(0 * NP, NP)], s1)
    pltpu.sync_copy(parta_h.at[pl.ds(3 * NP, NP)], acc)

    @pl.loop(0, NP, step=16)
    def _(i):
        sl = pl.ds(i, 16)
        d = s1[sl] + acc[sl]
        ae_v[sl] = ae_v[sl] / jnp.maximum(d, 1.0)

    _zero_1d(acc)

    @pl.loop(0, CJ)
    def _(j):
        for g in range(G16):
            sl = pl.ds(g * 16, 16)
            r = row_v[j, sl]
            c = col_v[j, sl]
            a0 = plsc.load_gather(ax_v, [r]) + plsc.load_gather(ae_v, [c])
            a = jnp.where(a0 > 0.0, a0, 0.2 * a0)
            e = jnp.exp(a)
            ea_v[j, sl] = e
            plsc.addupdate_scatter(acc, [c], e)

    pltpu.sync_copy(ea_v, ea_h.at[wid])
    pltpu.sync_copy(acc, shared.at[pl.ds(sid * NP, NP)])
    plsc.subcore_barrier()
    _reduce_tiles(shared, 0, NP, red, stg, rsem, ssum_h, cid * NP, sid)


@functools.partial(
    pl.kernel,
    out_type=(jax.ShapeDtypeStruct((NW, CJ, CK), _f32),
              jax.ShapeDtypeStruct((NW, CJ, CK), _f32)),
    mesh=_mesh,
    compiler_params=_sc_params,
    scratch_types=[
        pltpu.VMEM((CJ, CK), _i32),
        pltpu.VMEM((CJ, CK), _i32),
        pltpu.VMEM((CJ, CK), _f32),
        pltpu.VMEM((CJ, CK), _f32),
        pltpu.VMEM((CJ, CK), _f32),
        pltpu.VMEM((NP,), _f32),
        pltpu.VMEM((NP,), _f32),
        pltpu.VMEM((NP,), _f32),
    ],
)
def _sc_weights(row_h, col_h, ea_h, sb_h, qs_h, dn_h, w1_h, w2_h,
                row_v, col_v, ea_v, w1_v, w2_v, sb_v, qs_v, dn_v):
    cid, sid, wid = _ids()
    pltpu.sync_copy(row_h.at[wid], row_v)
    pltpu.sync_copy(col_h.at[wid], col_v)
    pltpu.sync_copy(ea_h.at[wid], ea_v)
    pltpu.sync_copy(sb_h, sb_v)
    pltpu.sync_copy(qs_h, qs_v)
    pltpu.sync_copy(dn_h, dn_v)

    @pl.loop(0, CJ)
    def _(j):
        for g in range(G16):
            sl = pl.ds(g * 16, 16)
            r = row_v[j, sl]
            c = col_v[j, sl]
            e = ea_v[j, sl]
            w1_v[j, sl] = e * plsc.load_gather(sb_v, [c])
            w2_v[j, sl] = (e * plsc.load_gather(dn_v, [r])
                           * plsc.load_gather(qs_v, [c]))

    pltpu.sync_copy(w1_v, w1_h.at[wid])
    pltpu.sync_copy(w2_v, w2_h.at[wid])


@functools.partial(
    pl.kernel,
    out_type=jax.ShapeDtypeStruct((NC, NP, D), _f32),
    mesh=_mesh,
    compiler_params=_sc_params,
    scratch_types=(
        [pltpu.VMEM((SCJ2, CK2), _i32),
         pltpu.VMEM((SCJ2, CK2), _i32),
         pltpu.VMEM((SCJ2, CK2), _f32)]
        + [pltpu.VMEM((CK2, D), _f32)] * NBUF
        + [pltpu.VMEM_SHARED((NP, D), _f32)]
        + [pltpu.SemaphoreType.DMA] * (2 * NBUF)
    ),
)
def _sc_prop(gi_h, si_h, w_h, tab_h, out_h, gi_v, si_v, w_v, *refs):
    bufs = refs[:NBUF]
    acc_sh = refs[NBUF]
    gsems = refs[NBUF + 1:NBUF + 1 + NBUF]
    ssems = refs[NBUF + 1 + NBUF:]
    cid, sid, wid = _ids()

    z = jnp.zeros((16,), _f32)

    @pl.loop(0, CK2)
    def _(rr):
        for k in range(D // 16):
            bufs[0][rr, pl.ds(k * 16, 16)] = z

    @pl.loop(0, SLC, step=CK2)
    def _(k):
        pltpu.sync_copy(bufs[0], acc_sh.at[pl.ds(sid * SLC + k, CK2)])

    plsc.subcore_barrier()

    def scale(buf, t):
        tsplat = jnp.full((16,), t, _i32)

        @pl.loop(0, CK2)
        def _(rr):
            wv = plsc.load_gather(w_v, [tsplat, jnp.full((16,), rr, _i32)])
            for k in range(D // 16):
                slk = pl.ds(k * 16, 16)
                buf[rr, slk] = buf[rr, slk] * wv

    @pl.loop(0, NSC2)
    def _(sc):
        scs = pl.ds(sc * SCJ2, SCJ2)
        pltpu.sync_copy(gi_h.at[wid, scs], gi_v)
        pltpu.sync_copy(si_h.at[wid, scs], si_v)
        pltpu.sync_copy(w_h.at[wid, scs], w_v)

        for p in range(NBUF):
            pltpu.async_copy(tab_h.at[gi_v.at[p]], bufs[p], gsems[p])

        @pl.loop(0, SCJ2, step=NBUF)
        def _(tt):
            for p in range(NBUF):
                cur = tt + p
                pltpu.make_async_copy(tab_h.at[gi_v.at[cur]], bufs[p],
                                      gsems[p]).wait()
                scale(bufs[p], cur)
                pltpu.async_copy(bufs[p], acc_sh.at[si_v.at[cur]],
                                 ssems[p], add=True)
                q = (p + 2) % NBUF
                nxt = cur + 2

                @pl.when(jnp.logical_and(nxt >= NBUF, nxt < SCJ2))
                def _():
                    pltpu.make_async_copy(bufs[q], acc_sh.at[si_v.at[0]],
                                          ssems[q]).wait()
                    pltpu.async_copy(tab_h.at[gi_v.at[nxt]], bufs[q],
                                     gsems[q])

        for p in range(NBUF):
            pltpu.make_async_copy(bufs[p], acc_sh.at[si_v.at[0]],
                                  ssems[p]).wait()

    plsc.subcore_barrier()
    pltpu.sync_copy(acc_sh.at[pl.ds(sid * SLC, SLC)],
                    out_h.at[cid, pl.ds(sid * SLC, SLC)])


def _lin_common(xb, w_ref, att_ref, xlin_ref, ax_ref, px_ref):
    W = w_ref[...]
    aF = att_ref[0, :]
    aH = att_ref[1, :]
    xl = lax.dot_general(xb, W, (((1,), (1,)), ((), ())),
                         preferred_element_type=_f32)
    u = jnp.dot(aH, W)
    xlin_ref[...] = xl
    ax_ref[...] = jnp.sum(xl * aF[None, :], axis=1)
    px_ref[...] = jnp.sum(xb * u[None, :], axis=1)


_LIN_OUT_SPECS = [
    pl.BlockSpec((BT, D), lambda i: (i, 0)),
    pl.BlockSpec((BT,), lambda i: (i,)),
    pl.BlockSpec((BT,), lambda i: (i,)),
]
_LIN_OUT_SHAPE = [
    jax.ShapeDtypeStruct((NP, D), _f32),
    jax.ShapeDtypeStruct((NP,), _f32),
    jax.ShapeDtypeStruct((NP,), _f32),
]


def _tc_lin_body(x_ref, w_ref, att_ref, xlin_ref, ax_ref, px_ref):
    _lin_common(x_ref[...], w_ref, att_ref, xlin_ref, ax_ref, px_ref)


def _tc_lin(x_pad, W, att2):
    return pl.pallas_call(
        _tc_lin_body,
        grid=(NP // BT,),
        in_specs=[
            pl.BlockSpec((BT, D), lambda i: (i, 0)),
            pl.BlockSpec((D, D), lambda i: (0, 0)),
            pl.BlockSpec((2, D), lambda i: (0, 0)),
        ],
        out_specs=_LIN_OUT_SPECS,
        out_shape=_LIN_OUT_SHAPE,
    )(x_pad, W, att2)


def _tc_tables_body(pa_ref, ss_ref, sb_ref, qs_ref, dn_ref):
    deg = pa_ref[0, 0] + pa_ref[1, 0]
    dn = pa_ref[0, 2] + pa_ref[1, 2]
    ss = ss_ref[0] + ss_ref[1]
    qs = 1.0 / (ss + 1e-16)
    qs_ref[...] = qs
    sb_ref[...] = jnp.where(deg > 0.0, qs / deg, 0.0)
    dn_ref[...] = jnp.where(dn > 0.0, 1.0 / dn, 0.0)


def _tc_tables(parta, ssum):
    pa = parta.reshape(NC, 3, NP)
    ss = ssum.reshape(NC, NP)
    return pl.pallas_call(
        _tc_tables_body,
        grid=(NP // BT,),
        in_specs=[
            pl.BlockSpec((NC, 3, BT), lambda i: (0, 0, i)),
            pl.BlockSpec((NC, BT), lambda i: (0, i)),
        ],
        out_specs=[pl.BlockSpec((BT,), lambda i: (i,))] * 3,
        out_shape=[jax.ShapeDtypeStruct((NP,), _f32)] * 3,
    )(pa, ss)


def _tc_sum_body(p_ref, o_ref):
    o_ref[...] = p_ref[0] + p_ref[1]


def _tc_sum(part):
    return pl.pallas_call(
        _tc_sum_body,
        grid=(NP // BT,),
        in_specs=[pl.BlockSpec((NC, BT, D), lambda i: (0, i, 0))],
        out_specs=pl.BlockSpec((BT, D), lambda i: (i, 0)),
        out_shape=jax.ShapeDtypeStruct((NP, D), _f32),
    )(part)


def _tc_mid_body(p_ref, b_ref, ap_ref, w_ref, att_ref,
                 xlin_ref, ax_ref, px_ref):
    ap = ap_ref[0, 0]
    t = p_ref[0] + p_ref[1] + b_ref[0, :][None, :]
    xb = jnp.where(t >= 0.0, t, ap * t)
    _lin_common(xb, w_ref, att_ref, xlin_ref, ax_ref, px_ref)


def _tc_mid(part, b, ap, W, att2):
    return pl.pallas_call(
        _tc_mid_body,
        grid=(NP // BT,),
        in_specs=[
            pl.BlockSpec((NC, BT, D), lambda i: (0, i, 0)),
            pl.BlockSpec((1, D), lambda i: (0, 0)),
            pl.BlockSpec((1, 1), lambda i: (0, 0)),
            pl.BlockSpec((D, D), lambda i: (0, 0)),
            pl.BlockSpec((2, D), lambda i: (0, 0)),
        ],
        out_specs=_LIN_OUT_SPECS,
        out_shape=_LIN_OUT_SHAPE,
    )(part, b, ap, W, att2)


def _tc_final_body(p_ref, b_ref, x_ref, ap_ref, o_ref):
    ap = ap_ref[0, 0]
    t = p_ref[0] + p_ref[1] + b_ref[0, :][None, :] + x_ref[...]
    o_ref[...] = jnp.where(t >= 0.0, t, ap * t)


def _tc_final(part, b, x_pad, ap):
    return pl.pallas_call(
        _tc_final_body,
        grid=(NP // BT,),
        in_specs=[
            pl.BlockSpec((NC, BT, D), lambda i: (0, i, 0)),
            pl.BlockSpec((1, D), lambda i: (0, 0)),
            pl.BlockSpec((BT, D), lambda i: (i, 0)),
            pl.BlockSpec((1, 1), lambda i: (0, 0)),
        ],
        out_specs=pl.BlockSpec((BT, D), lambda i: (i, 0)),
        out_shape=jax.ShapeDtypeStruct((NP, D), _f32),
    )(part, b, x_pad, ap)


def kernel(x, edge_index, W1, att1, b1, W2, att2, b2, a_prelu):
    epw = E // NW
    padw = EW - epw
    padi = jnp.broadcast_to(
        (TRB + jnp.arange(NW, dtype=_i32))[:, None], (NW, padw))
    row3 = jnp.concatenate(
        [edge_index[0].reshape(NW, epw), padi], axis=1).reshape(NW, CJ, CK)
    col3 = jnp.concatenate(
        [edge_index[1].reshape(NW, epw), padi], axis=1).reshape(NW, CJ, CK)
    x_pad = jnp.pad(x, ((0, NP - N), (0, 0)))
    att1_2 = att1.reshape(2, D)
    att2_2 = att2.reshape(2, D)
    b1_2 = b1.reshape(1, D)
    b2_2 = b2.reshape(1, D)
    ap = a_prelu.reshape(1, 1)

    def layer(xlin, ax, px):
        parta = _sc_stats(row3, col3, px)
        ea, ssum = _sc_soft(row3, col3, ax, parta)
        sb, qs, dninv = _tc_tables(parta, ssum)
        w1, w2 = _sc_weights(row3, col3, ea, sb, qs, dninv)
        shp = (NW, SCJ2 * NSC2, CK2)
        parte = _sc_prop(row3.reshape(shp), col3.reshape(shp),
                         w1.reshape(shp), xlin)
        oute = _tc_sum(parte)
        return _sc_prop(col3.reshape(shp), row3.reshape(shp),
                        w2.reshape(shp), oute)

    xlin1, ax1, px1 = _tc_lin(x_pad, W1, att1_2)
    partn1 = layer(xlin1, ax1, px1)
    xlin2, ax2, px2 = _tc_mid(partn1, b1_2, ap, W2, att2_2)
    partn2 = layer(xlin2, ax2, px2)
    out = _tc_final(partn2, b2_2, x_pad, ap)
    return out[:N]
